# baseline (speedup 1.0000x reference)
import jax
import jax.numpy as jnp
from jax.experimental import pallas as pl
from jax.experimental.pallas import tpu as pltpu

_B, _N, _M, _C = 4, 120000, 32, 80
_TN = 15000
_NB = _N // _TN
_S = _B * _NB

_INTERPRET = False


def _loss_kernel(cls_hbm, bbox_hbm, anc_hbm, box_ref, ext_ref, out_ref,
                 bufc, bufb, bufa, acc_ref, sems):
    b = pl.program_id(0)
    i = pl.program_id(1)
    s = b * _NB + i
    cur = jax.lax.rem(s, 2)
    nxt = 1 - cur

    def start_copies(bb, ii, slot):
        pltpu.make_async_copy(cls_hbm.at[bb, pl.ds(ii * _TN, _TN), :],
                              bufc.at[slot], sems.at[0, slot]).start()
        pltpu.make_async_copy(bbox_hbm.at[bb, pl.ds(ii * _TN, _TN), :],
                              bufb.at[slot], sems.at[1, slot]).start()
        pltpu.make_async_copy(anc_hbm.at[bb, pl.ds(ii * _TN, _TN), :],
                              bufa.at[slot], sems.at[2, slot]).start()

    @pl.when(s == 0)
    def _first():
        start_copies(0, 0, 0)

    @pl.when(s + 1 < _S)
    def _prefetch():
        start_copies((s + 1) // _NB, (s + 1) % _NB, nxt)

    pltpu.make_async_copy(cls_hbm.at[b, pl.ds(i * _TN, _TN), :],
                          bufc.at[cur], sems.at[0, cur]).wait()
    pltpu.make_async_copy(bbox_hbm.at[b, pl.ds(i * _TN, _TN), :],
                          bufb.at[cur], sems.at[1, cur]).wait()
    pltpu.make_async_copy(anc_hbm.at[b, pl.ds(i * _TN, _TN), :],
                          bufa.at[cur], sems.at[2, cur]).wait()

    @pl.when(i == 0)
    def _init():
        acc_ref[b, 0] = 0.0
        acc_ref[b, 1] = 0.0
        acc_ref[b, 2] = 0.0

    anct = jax.lax.transpose(bufa[cur], (1, 0))
    ax0 = anct[0:1, :]
    ay0 = anct[1:2, :]
    ax1 = anct[2:3, :]
    ay1 = anct[3:4, :]

    boxes_blk = box_ref[0]
    bx0 = boxes_blk[:, 0:1]
    by0 = boxes_blk[:, 1:2]
    bx1 = boxes_blk[:, 2:3]
    by1 = boxes_blk[:, 3:4]
    ext = ext_ref[0]

    iw = jnp.maximum(jnp.minimum(ax1, bx1) - jnp.maximum(ax0, bx0), 0.0)
    ih = jnp.maximum(jnp.minimum(ay1, by1) - jnp.maximum(ay0, by0), 0.0)
    inter = iw * ih
    area_a = (ax1 - ax0) * (ay1 - ay0)
    area_b = (bx1 - bx0) * (by1 - by0)
    union = area_a + area_b - inter
    iou = inter / jnp.maximum(union, 1e-9)

    vals = jnp.max(iou, axis=0, keepdims=True)
    ids = jax.lax.broadcasted_iota(jnp.int32, (_M, _TN), 0)
    cand = jnp.where(iou >= vals, ids, _M)
    idxm = jnp.min(cand, axis=0, keepdims=True)
    onehot = (ids == idxm).astype(jnp.float32)

    pos = vals >= 0.5
    posf = pos.astype(jnp.float32)
    ignore = jnp.logical_and(vals >= 0.4, vals < 0.5)
    maskf = jnp.where(ignore, 0.0, 1.0)

    sel8 = jax.lax.dot_general(ext, onehot, (((1,), (0,)), ((), ())),
                               preferred_element_type=jnp.float32)
    sx0 = sel8[0:1, :]
    sy0 = sel8[1:2, :]
    sx1 = sel8[2:3, :]
    sy1 = sel8[3:4, :]
    slab = sel8[4:5, :]

    scx = (sx0 + sx1) * 0.5
    scy = (sy0 + sy1) * 0.5
    sw = sx1 - sx0
    sh = sy1 - sy0
    acx = (ax0 + ax1) * 0.5
    acy = (ay0 + ay1) * 0.5
    aw = jnp.maximum(ax1 - ax0, 1e-9)
    ah = jnp.maximum(ay1 - ay0, 1e-9)
    tx = ((scx - acx) / aw) / 0.1
    ty = ((scy - acy) / ah) / 0.1
    tw = jnp.log(jnp.maximum(sw, 1e-9) / aw) / 0.2
    th = jnp.log(jnp.maximum(sh, 1e-9) / ah) / 0.2

    bpt = jax.lax.transpose(bufb[cur], (1, 0))
    sl1 = jnp.zeros((1, _TN), jnp.float32)
    for kk, enc in enumerate((tx, ty, tw, th)):
        d = bpt[kk:kk + 1, :] - enc
        ad = jnp.abs(d)
        sl1 = sl1 + jnp.where(ad < 1.0, 0.5 * d * d, ad - 0.5)
    bb_par = jnp.sum(sl1 * posf)
    np_par = jnp.sum(posf)

    xt = jax.lax.transpose(bufc[cur], (1, 0))
    e = jnp.exp(-jnp.abs(xt))
    r = 1.0 / (1.0 + e)
    ps = jnp.where(xt >= 0.0, r, 1.0 - r)
    sp = jnp.maximum(xt, 0.0) + jnp.log1p(e)
    f0 = ps * ps * sp
    f_rows = jnp.sum(f0, axis=0, keepdims=True) * maskf

    cio = jax.lax.broadcasted_iota(jnp.int32, (_C, _TN), 0)
    slabi = slab.astype(jnp.int32)
    xl = jnp.sum(jnp.where(cio == slabi - 1, xt, 0.0), axis=0,
                 keepdims=True)
    el = jnp.exp(-jnp.abs(xl))
    rl = 1.0 / (1.0 + el)
    psl = jnp.where(xl >= 0.0, rl, 1.0 - rl)
    spl = jnp.maximum(xl, 0.0) + jnp.log1p(el)
    f0l = 0.25 * psl * psl * spl
    f1l = 0.75 * (1.0 - psl) * (1.0 - psl) * (spl - xl)
    foc_par = 0.25 * jnp.sum(f_rows) + jnp.sum((f1l - f0l) * posf)

    acc_ref[b, 0] = acc_ref[b, 0] + foc_par
    acc_ref[b, 1] = acc_ref[b, 1] + bb_par
    acc_ref[b, 2] = acc_ref[b, 2] + np_par

    @pl.when(jnp.logical_and(b == _B - 1, i == _NB - 1))
    def _fin():
        cl = 0.0
        rl2 = 0.0
        for bb in range(_B):
            npos = acc_ref[bb, 2]
            cl = cl + acc_ref[bb, 0] / jnp.maximum(npos, 1.0)
            rl2 = rl2 + acc_ref[bb, 1] / jnp.maximum(npos * 4.0, 1.0)
        out_ref[0, 0] = cl / _B
        out_ref[0, 1] = rl2 / _B


def kernel(cls_preds, bbox_preds, anchors, boxes, labels):
    ext = jnp.concatenate(
        [jnp.transpose(boxes, (0, 2, 1)),
         labels.astype(jnp.float32)[:, None, :],
         jnp.zeros((_B, 3, _M), jnp.float32)], axis=1)

    out = pl.pallas_call(
        _loss_kernel,
        grid=(_B, _NB),
        in_specs=[
            pl.BlockSpec(memory_space=pltpu.MemorySpace.HBM),
            pl.BlockSpec(memory_space=pltpu.MemorySpace.HBM),
            pl.BlockSpec(memory_space=pltpu.MemorySpace.HBM),
            pl.BlockSpec((1, _M, 4), lambda b, i: (b, 0, 0)),
            pl.BlockSpec((1, 8, _M), lambda b, i: (b, 0, 0)),
        ],
        out_specs=pl.BlockSpec((1, 2), lambda b, i: (0, 0), memory_space=pltpu.SMEM),
        out_shape=jax.ShapeDtypeStruct((1, 2), jnp.float32),
        scratch_shapes=[
            pltpu.VMEM((2, _TN, _C), jnp.float32),
            pltpu.VMEM((2, _TN, 4), jnp.float32),
            pltpu.VMEM((2, _TN, 4), jnp.float32),
            pltpu.SMEM((_B, 3), jnp.float32),
            pltpu.SemaphoreType.DMA((3, 2)),
        ],
        interpret=_INTERPRET,
    )(cls_preds, bbox_preds, anchors, boxes, ext)
    return out[0, 0], out[0, 1]


# speedup vs baseline: 1.5179x; 1.5179x over previous
import jax
import jax.numpy as jnp
from jax.experimental import pallas as pl
from jax.experimental.pallas import tpu as pltpu

_B, _N, _M, _C = 4, 120000, 32, 80
_TN = 15000
_NB = _N // _TN

_INTERPRET = False


def _loss_kernel(cls_ref, bbox_ref, anc_ref, box_ref, ext_ref, out_ref, acc_ref):
    b = pl.program_id(0)
    i = pl.program_id(1)

    @pl.when(i == 0)
    def _init():
        acc_ref[b, 0] = 0.0
        acc_ref[b, 1] = 0.0
        acc_ref[b, 2] = 0.0

    anc = anc_ref[0]
    ax0 = anc[0:1, :]
    ay0 = anc[1:2, :]
    ax1 = anc[2:3, :]
    ay1 = anc[3:4, :]

    boxes_blk = box_ref[0]
    bx0 = boxes_blk[:, 0:1]
    by0 = boxes_blk[:, 1:2]
    bx1 = boxes_blk[:, 2:3]
    by1 = boxes_blk[:, 3:4]
    ext = ext_ref[0]

    iw = jnp.maximum(jnp.minimum(ax1, bx1) - jnp.maximum(ax0, bx0), 0.0)
    ih = jnp.maximum(jnp.minimum(ay1, by1) - jnp.maximum(ay0, by0), 0.0)
    inter = iw * ih
    area_a = (ax1 - ax0) * (ay1 - ay0)
    area_b = (bx1 - bx0) * (by1 - by0)
    union = area_a + area_b - inter
    iou = inter / jnp.maximum(union, 1e-9)

    vals = jnp.max(iou, axis=0, keepdims=True)
    ids = jax.lax.broadcasted_iota(jnp.int32, (_M, _TN), 0)
    cand = jnp.where(iou >= vals, ids, _M)
    idxm = jnp.min(cand, axis=0, keepdims=True)
    onehot = (ids == idxm).astype(jnp.float32)

    pos = vals >= 0.5
    posf = pos.astype(jnp.float32)
    ignore = jnp.logical_and(vals >= 0.4, vals < 0.5)
    maskf = jnp.where(ignore, 0.0, 1.0)

    sel8 = jax.lax.dot_general(ext, onehot, (((1,), (0,)), ((), ())),
                               preferred_element_type=jnp.float32)
    sx0 = sel8[0:1, :]
    sy0 = sel8[1:2, :]
    sx1 = sel8[2:3, :]
    sy1 = sel8[3:4, :]
    slab = sel8[4:5, :]

    scx = (sx0 + sx1) * 0.5
    scy = (sy0 + sy1) * 0.5
    sw = sx1 - sx0
    sh = sy1 - sy0
    acx = (ax0 + ax1) * 0.5
    acy = (ay0 + ay1) * 0.5
    aw = jnp.maximum(ax1 - ax0, 1e-9)
    ah = jnp.maximum(ay1 - ay0, 1e-9)
    tx = ((scx - acx) / aw) / 0.1
    ty = ((scy - acy) / ah) / 0.1
    tw = jnp.log(jnp.maximum(sw, 1e-9) / aw) / 0.2
    th = jnp.log(jnp.maximum(sh, 1e-9) / ah) / 0.2

    bp = bbox_ref[0]
    sl1 = jnp.zeros((1, _TN), jnp.float32)
    for kk, enc in enumerate((tx, ty, tw, th)):
        d = bp[kk:kk + 1, :] - enc
        ad = jnp.abs(d)
        sl1 = sl1 + jnp.where(ad < 1.0, 0.5 * d * d, ad - 0.5)
    bb_par = jnp.sum(sl1 * posf)
    np_par = jnp.sum(posf)

    xt = jax.lax.transpose(cls_ref[0], (1, 0))
    cio = jax.lax.broadcasted_iota(jnp.int32, (_C, _TN), 0)
    slabi = slab.astype(jnp.int32)
    e = jnp.exp(-jnp.abs(xt))
    r = 1.0 / (1.0 + e)
    ps = jnp.where(xt >= 0.0, r, 1.0 - r)
    sp = jnp.maximum(xt, 0.0) + jnp.log1p(e)
    f0 = ps * ps * sp
    f_rows = jnp.sum(f0, axis=0, keepdims=True) * maskf
    xl = jnp.sum(jnp.where(cio == slabi - 1, xt, 0.0), axis=0,
                 keepdims=True)

    el = jnp.exp(-jnp.abs(xl))
    rl = 1.0 / (1.0 + el)
    psl = jnp.where(xl >= 0.0, rl, 1.0 - rl)
    spl = jnp.maximum(xl, 0.0) + jnp.log1p(el)
    f0l = 0.25 * psl * psl * spl
    f1l = 0.75 * (1.0 - psl) * (1.0 - psl) * (spl - xl)
    foc_par = 0.25 * jnp.sum(f_rows) + jnp.sum((f1l - f0l) * posf)

    acc_ref[b, 0] = acc_ref[b, 0] + foc_par
    acc_ref[b, 1] = acc_ref[b, 1] + bb_par
    acc_ref[b, 2] = acc_ref[b, 2] + np_par

    @pl.when(jnp.logical_and(b == _B - 1, i == _NB - 1))
    def _fin():
        cl = 0.0
        rl2 = 0.0
        for bb in range(_B):
            npos = acc_ref[bb, 2]
            cl = cl + acc_ref[bb, 0] / jnp.maximum(npos, 1.0)
            rl2 = rl2 + acc_ref[bb, 1] / jnp.maximum(npos * 4.0, 1.0)
        out_ref[0, 0] = cl / _B
        out_ref[0, 1] = rl2 / _B


def _retile(a):
    a = jnp.transpose(a, (0, 2, 1))
    a = a.reshape(_B, 4, _NB, _TN)
    a = jnp.transpose(a, (0, 2, 1, 3))
    return a.reshape(_B * _NB, 4, _TN)


def kernel(cls_preds, bbox_preds, anchors, boxes, labels):
    anc_r = _retile(anchors)
    bbox_r = _retile(bbox_preds)
    ext = jnp.concatenate(
        [jnp.transpose(boxes, (0, 2, 1)),
         labels.astype(jnp.float32)[:, None, :],
         jnp.zeros((_B, 3, _M), jnp.float32)], axis=1)

    out = pl.pallas_call(
        _loss_kernel,
        grid=(_B, _NB),
        in_specs=[
            pl.BlockSpec((1, _TN, _C), lambda b, i: (b, i, 0)),
            pl.BlockSpec((1, 4, _TN), lambda b, i: (b * _NB + i, 0, 0)),
            pl.BlockSpec((1, 4, _TN), lambda b, i: (b * _NB + i, 0, 0)),
            pl.BlockSpec((1, _M, 4), lambda b, i: (b, 0, 0)),
            pl.BlockSpec((1, 8, _M), lambda b, i: (b, 0, 0)),
        ],
        out_specs=pl.BlockSpec((1, 2), lambda b, i: (0, 0), memory_space=pltpu.SMEM),
        out_shape=jax.ShapeDtypeStruct((1, 2), jnp.float32),
        scratch_shapes=[pltpu.SMEM((_B, 3), jnp.float32)],
        interpret=_INTERPRET,
    )(cls_preds, bbox_r, anc_r, boxes, ext)
    return out[0, 0], out[0, 1]
